# baseline (speedup 1.0000x reference)
import functools

import jax
import jax.numpy as jnp
from jax import lax
from jax.experimental import pallas as pl
from jax.experimental.pallas import tpu as pltpu
from jax.experimental.pallas import tpu_sc as plsc

N = 10000
E = 320000
NFEAT = 128
NLABEL = 16
NHID = 128

NC, NS, LANES = 2, 16, 16
NW = NC * NS
NPAD = 10240
CHUNK = 72
NCHUNK = (E + NW * CHUNK - 1) // (NW * CHUNK)
EPAD = NW * NCHUNK * CHUNK
RPT = NPAD // NS
NIDXL = 1008


def _sc_deg_feats_body(dst_hbm, idxl_hbm, ones_hbm, z_hbm,
                       deg_out, mark_out,
                       dst_v, idxl_v, ones_v, z_v, deg_sp, mark_sp, sem):
    c = lax.axis_index("c")
    s = lax.axis_index("s")
    wid = s * NC + c

    pltpu.sync_copy(z_hbm, z_v)
    pltpu.sync_copy(z_v, deg_sp.at[pl.ds(s * RPT, RPT)])
    pltpu.sync_copy(z_v, mark_sp.at[pl.ds(s * RPT, RPT)])
    plsc.subcore_barrier()

    pltpu.sync_copy(ones_hbm, ones_v)
    pltpu.sync_copy(dst_hbm.at[wid], dst_v)
    pltpu.sync_copy(idxl_hbm, idxl_v)

    def chunk(j, carry):
        pltpu.async_copy(ones_v, deg_sp.at[dst_v.at[j]], sem, add=True)
        return carry

    lax.fori_loop(0, NCHUNK, chunk, 0)

    for j in range(NIDXL // CHUNK):
        pltpu.async_copy(ones_v, mark_sp.at[idxl_v.at[j]], sem, add=True)

    def drain(j, carry):
        pltpu.make_async_copy(ones_v, deg_sp.at[dst_v.at[j]], sem).wait()
        return carry

    lax.fori_loop(0, NCHUNK, drain, 0)
    for j in range(NIDXL // CHUNK):
        pltpu.make_async_copy(ones_v, mark_sp.at[idxl_v.at[j]], sem).wait()

    plsc.subcore_barrier()
    pltpu.sync_copy(deg_sp.at[pl.ds(s * RPT, RPT)],
                    deg_out.at[pl.ds((c * NS + s) * RPT, RPT)])
    pltpu.sync_copy(mark_sp.at[pl.ds(s * RPT, RPT)],
                    mark_out.at[pl.ds((c * NS + s) * RPT, RPT)])


@functools.cache
def _get_sc_deg_feats():
  return pl.kernel(
    _sc_deg_feats_body,
    out_type=(jax.ShapeDtypeStruct((NC * NPAD, LANES), jnp.float32),
              jax.ShapeDtypeStruct((NC * NPAD, LANES), jnp.float32)),
    mesh=plsc.VectorSubcoreMesh(core_axis_name="c", subcore_axis_name="s",
                                num_cores=NC, num_subcores=NS),
    compiler_params=pltpu.CompilerParams(use_tc_tiling_on_sc=False),
    scratch_types=[
        pltpu.VMEM((NCHUNK, CHUNK), jnp.int32),
        pltpu.VMEM((NIDXL // CHUNK, CHUNK), jnp.int32),
        pltpu.VMEM((CHUNK, LANES), jnp.float32),
        pltpu.VMEM((RPT, LANES), jnp.float32),
        pltpu.VMEM_SHARED((NPAD, LANES), jnp.float32),
        pltpu.VMEM_SHARED((NPAD, LANES), jnp.float32),
        pltpu.SemaphoreType.DMA,
    ],
  )


DEPTH = 3


def _sc_spmm_body(hws_hbm, src_hbm, dst_hbm, z_hbm, out_hbm,
                  src_v, dst_v, rows_v, acc_sp, gsem, ssem):
    c = lax.axis_index("c")
    s = lax.axis_index("s")
    wid = s * NC + c

    pltpu.sync_copy(z_hbm, acc_sp.at[pl.ds(s * RPT, RPT)])
    plsc.subcore_barrier()

    pltpu.sync_copy(src_hbm.at[wid], src_v)
    pltpu.sync_copy(dst_hbm.at[wid], dst_v)

    for p in range(DEPTH - 1):
        pltpu.async_copy(hws_hbm.at[src_v.at[p]], rows_v.at[p], gsem)

    def chunk(j, carry):
        @pl.when(j >= 1)
        def _():
            pltpu.make_async_copy(rows_v.at[(j - 1) % DEPTH],
                                  acc_sp.at[dst_v.at[j - 1]], ssem).wait()

        nxt = j + DEPTH - 1

        @pl.when(nxt < NCHUNK)
        def _():
            pltpu.async_copy(hws_hbm.at[src_v.at[nxt]],
                             rows_v.at[nxt % DEPTH], gsem)

        pltpu.make_async_copy(hws_hbm.at[src_v.at[j]], rows_v.at[j % DEPTH],
                              gsem).wait()
        pltpu.async_copy(rows_v.at[j % DEPTH], acc_sp.at[dst_v.at[j]], ssem,
                         add=True)
        return carry

    lax.fori_loop(0, NCHUNK, chunk, 0)
    pltpu.make_async_copy(rows_v.at[(NCHUNK - 1) % DEPTH],
                          acc_sp.at[dst_v.at[NCHUNK - 1]], ssem).wait()

    plsc.subcore_barrier()
    pltpu.sync_copy(acc_sp.at[pl.ds(s * RPT, RPT)],
                    out_hbm.at[pl.ds((c * NS + s) * RPT, RPT)])


@functools.cache
def _get_sc_spmm():
  return pl.kernel(
    _sc_spmm_body,
    out_type=jax.ShapeDtypeStruct((NC * NPAD, NHID), jnp.float32),
    mesh=plsc.VectorSubcoreMesh(core_axis_name="c", subcore_axis_name="s",
                                num_cores=NC, num_subcores=NS),
    compiler_params=pltpu.CompilerParams(use_tc_tiling_on_sc=False),
    scratch_types=[
        pltpu.VMEM((NCHUNK, CHUNK), jnp.int32),
        pltpu.VMEM((NCHUNK, CHUNK), jnp.int32),
        pltpu.VMEM((DEPTH, CHUNK, NHID), jnp.float32),
        pltpu.VMEM_SHARED((NPAD, NHID), jnp.float32),
        pltpu.SemaphoreType.DMA,
        pltpu.SemaphoreType.DMA,
    ],
  )


BM = 512
GRID = NPAD // BM


def _tc0_body(x_ref, w1a_ref, hwx_ref):
    hwx_ref[...] = jnp.dot(x_ref[...], w1a_ref[...],
                           preferred_element_type=jnp.float32)


_tc0 = pl.pallas_call(
    _tc0_body,
    grid=(GRID,),
    in_specs=[
        pl.BlockSpec((BM, NFEAT), lambda i: (i, 0)),
        pl.BlockSpec((NFEAT, NHID), lambda i: (0, 0)),
    ],
    out_specs=pl.BlockSpec((BM, NHID), lambda i: (i, 0)),
    out_shape=jax.ShapeDtypeStruct((NPAD, NHID), jnp.float32),
)


def _tc1_body(hwx_ref, y_ref, mark_ref, dega_ref, degb_ref, w1b_ref,
              hws_ref, sl_ref, dinv_ref, dself_ref):
    deg = dega_ref[...] + degb_ref[...] + 2.0
    dinv = lax.rsqrt(deg)
    dself = 2.0 / deg
    oh = (y_ref[...] == lax.broadcasted_iota(
        jnp.int32, (BM, NLABEL), 1)).astype(jnp.float32)
    oh = oh * jnp.minimum(mark_ref[...], 1.0)
    hw = hwx_ref[...] + jnp.dot(oh, w1b_ref[...],
                                preferred_element_type=jnp.float32)
    hws_ref[...] = dinv * hw
    sl_ref[...] = dself * hw
    dinv_ref[...] = dinv
    dself_ref[...] = dself


_tc1 = pl.pallas_call(
    _tc1_body,
    grid=(GRID,),
    in_specs=[
        pl.BlockSpec((BM, NHID), lambda i: (i, 0)),
        pl.BlockSpec((BM, 1), lambda i: (i, 0)),
        pl.BlockSpec((BM, 1), lambda i: (i, 0)),
        pl.BlockSpec((BM, 1), lambda i: (i, 0)),
        pl.BlockSpec((BM, 1), lambda i: (i, 0)),
        pl.BlockSpec((NLABEL, NHID), lambda i: (0, 0)),
    ],
    out_specs=[
        pl.BlockSpec((BM, NHID), lambda i: (i, 0)),
        pl.BlockSpec((BM, NHID), lambda i: (i, 0)),
        pl.BlockSpec((BM, 1), lambda i: (i, 0)),
        pl.BlockSpec((BM, 1), lambda i: (i, 0)),
    ],
    out_shape=[
        jax.ShapeDtypeStruct((NPAD, NHID), jnp.float32),
        jax.ShapeDtypeStruct((NPAD, NHID), jnp.float32),
        jax.ShapeDtypeStruct((NPAD, 1), jnp.float32),
        jax.ShapeDtypeStruct((NPAD, 1), jnp.float32),
    ],
)


def _tc2_body(acca_ref, accb_ref, sl_ref, dinv_ref, dself_ref, b_ref, w_ref,
              hws_ref, slo_ref):
    dinv = dinv_ref[...]
    h = dinv * (acca_ref[...] + accb_ref[...]) + sl_ref[...] + b_ref[...]
    h = jnp.maximum(h, 0.0)
    hw = jnp.dot(h, w_ref[...], preferred_element_type=jnp.float32)
    hws_ref[...] = dinv * hw
    slo_ref[...] = dself_ref[...] * hw


_tc2 = pl.pallas_call(
    _tc2_body,
    grid=(GRID,),
    in_specs=[
        pl.BlockSpec((BM, NHID), lambda i: (i, 0)),
        pl.BlockSpec((BM, NHID), lambda i: (i, 0)),
        pl.BlockSpec((BM, NHID), lambda i: (i, 0)),
        pl.BlockSpec((BM, 1), lambda i: (i, 0)),
        pl.BlockSpec((BM, 1), lambda i: (i, 0)),
        pl.BlockSpec((1, NHID), lambda i: (0, 0)),
        pl.BlockSpec((NHID, NHID), lambda i: (0, 0)),
    ],
    out_specs=[
        pl.BlockSpec((BM, NHID), lambda i: (i, 0)),
        pl.BlockSpec((BM, NHID), lambda i: (i, 0)),
    ],
    out_shape=[
        jax.ShapeDtypeStruct((NPAD, NHID), jnp.float32),
        jax.ShapeDtypeStruct((NPAD, NHID), jnp.float32),
    ],
)


def _tc3_body(acca_ref, accb_ref, sl_ref, dinv_ref, b2_ref,
              wf1_ref, bf1_ref, wf2_ref, bf2_ref, out_ref):
    h = dinv_ref[...] * (acca_ref[...] + accb_ref[...]) \
        + sl_ref[...] + b2_ref[...]
    h = jnp.maximum(h, 0.0)
    m = jnp.dot(h, wf1_ref[...], preferred_element_type=jnp.float32) \
        + bf1_ref[...]
    m = jnp.where(m > 0.0, m, jnp.exp(jnp.minimum(m, 0.0)) - 1.0)
    logits = jnp.dot(m, wf2_ref[...], preferred_element_type=jnp.float32) \
        + bf2_ref[...]
    mx = jnp.max(logits, axis=1, keepdims=True)
    sh = logits - mx
    lse = jnp.log(jnp.sum(jnp.exp(sh), axis=1, keepdims=True))
    out_ref[...] = sh - lse


_tc3 = pl.pallas_call(
    _tc3_body,
    grid=(GRID,),
    in_specs=[
        pl.BlockSpec((BM, NHID), lambda i: (i, 0)),
        pl.BlockSpec((BM, NHID), lambda i: (i, 0)),
        pl.BlockSpec((BM, NHID), lambda i: (i, 0)),
        pl.BlockSpec((BM, 1), lambda i: (i, 0)),
        pl.BlockSpec((1, NHID), lambda i: (0, 0)),
        pl.BlockSpec((NHID, 2 * NHID), lambda i: (0, 0)),
        pl.BlockSpec((1, 2 * NHID), lambda i: (0, 0)),
        pl.BlockSpec((2 * NHID, NLABEL), lambda i: (0, 0)),
        pl.BlockSpec((1, NLABEL), lambda i: (0, 0)),
    ],
    out_specs=pl.BlockSpec((BM, NLABEL), lambda i: (i, 0)),
    out_shape=jax.ShapeDtypeStruct((NPAD, NLABEL), jnp.float32),
)


def kernel(x, y, predictions, adj, idx_labeled, n_sample,
           W1, b1, W2, b2, Wf1, bf1, Wf2, bf2):
    src = adj[0]
    dst = adj[1]
    pad_i = jnp.arange(EPAD - E, dtype=jnp.int32)
    src_p = jnp.concatenate(
        [src, pad_i % N]).reshape(NW, NCHUNK, CHUNK)
    dst_p = jnp.concatenate(
        [dst, N + pad_i % (NPAD - N)]).reshape(NW, NCHUNK, CHUNK)
    idxl_p = jnp.concatenate(
        [idx_labeled, jnp.full((NIDXL - idx_labeled.shape[0],), NPAD - 1,
                               jnp.int32)]).reshape(NIDXL // CHUNK, CHUNK)
    y_p = jnp.concatenate([y, jnp.zeros((NPAD - N,), jnp.int32)])
    x_p = jnp.pad(x, ((0, NPAD - N), (0, 0)))
    ones_le = jnp.ones((CHUNK, LANES), jnp.float32)
    z_le = jnp.zeros((RPT, LANES), jnp.float32)
    z_row = jnp.zeros((RPT, NHID), jnp.float32)

    hwx = _tc0(x_p, W1[:NFEAT])

    deg_part, mark = _get_sc_deg_feats()(dst_p, idxl_p, ones_le, z_le)
    dega = deg_part[:NPAD, 0:1]
    degb = deg_part[NPAD:, 0:1]
    mark_col = mark[:NPAD, 0:1] + mark[NPAD:, 0:1]

    hws1, sl1, dinv, dself = _tc1(
        hwx, y_p.reshape(NPAD, 1), mark_col, dega, degb, W1[NFEAT:])

    acc1 = _get_sc_spmm()(hws1, src_p, dst_p, z_row)

    hws2, sl2 = _tc2(acc1[:NPAD], acc1[NPAD:], sl1, dinv, dself,
                     b1.reshape(1, NHID), W2)

    acc2 = _get_sc_spmm()(hws2, src_p, dst_p, z_row)

    out = _tc3(acc2[:NPAD], acc2[NPAD:], sl2, dinv, b2.reshape(1, NHID),
               Wf1, bf1.reshape(1, 2 * NHID), Wf2, bf2.reshape(1, NLABEL))
    return out[:N]


# speedup vs baseline: 1.0039x; 1.0039x over previous
import functools

import jax
import jax.numpy as jnp
from jax import lax
from jax.experimental import pallas as pl
from jax.experimental.pallas import tpu as pltpu
from jax.experimental.pallas import tpu_sc as plsc

N = 10000
E = 320000
NFEAT = 128
NLABEL = 16
NHID = 128

NC, NS, LANES = 2, 16, 16
NW = NC * NS
NPAD = 10240
CHUNK = 72
NCHUNK = (E + NW * CHUNK - 1) // (NW * CHUNK)
EPAD = NW * NCHUNK * CHUNK
RPT = NPAD // NS
NIDXL = 1008


def _sc_deg_feats_body(dst_hbm, idxl_hbm, ones_hbm, z_hbm,
                       deg_out, mark_out,
                       dst_v, idxl_v, ones_v, z_v, deg_sp, mark_sp, sem):
    c = lax.axis_index("c")
    s = lax.axis_index("s")
    wid = s * NC + c

    pltpu.sync_copy(z_hbm, z_v)
    pltpu.sync_copy(z_v, deg_sp.at[pl.ds(s * RPT, RPT)])
    pltpu.sync_copy(z_v, mark_sp.at[pl.ds(s * RPT, RPT)])
    plsc.subcore_barrier()

    pltpu.sync_copy(ones_hbm, ones_v)
    pltpu.sync_copy(dst_hbm.at[wid], dst_v)
    pltpu.sync_copy(idxl_hbm, idxl_v)

    def chunk(j, carry):
        pltpu.async_copy(ones_v, deg_sp.at[dst_v.at[j]], sem, add=True)
        return carry

    lax.fori_loop(0, NCHUNK, chunk, 0)

    for j in range(NIDXL // CHUNK):
        pltpu.async_copy(ones_v, mark_sp.at[idxl_v.at[j]], sem, add=True)

    def drain(j, carry):
        pltpu.make_async_copy(ones_v, deg_sp.at[dst_v.at[j]], sem).wait()
        return carry

    lax.fori_loop(0, NCHUNK, drain, 0)
    for j in range(NIDXL // CHUNK):
        pltpu.make_async_copy(ones_v, mark_sp.at[idxl_v.at[j]], sem).wait()

    plsc.subcore_barrier()
    pltpu.sync_copy(deg_sp.at[pl.ds(s * RPT, RPT)],
                    deg_out.at[pl.ds((c * NS + s) * RPT, RPT)])
    pltpu.sync_copy(mark_sp.at[pl.ds(s * RPT, RPT)],
                    mark_out.at[pl.ds((c * NS + s) * RPT, RPT)])


@functools.cache
def _get_sc_deg_feats():
  return pl.kernel(
    _sc_deg_feats_body,
    out_type=(jax.ShapeDtypeStruct((NC * NPAD, LANES), jnp.float32),
              jax.ShapeDtypeStruct((NC * NPAD, LANES), jnp.float32)),
    mesh=plsc.VectorSubcoreMesh(core_axis_name="c", subcore_axis_name="s",
                                num_cores=NC, num_subcores=NS),
    compiler_params=pltpu.CompilerParams(use_tc_tiling_on_sc=False),
    scratch_types=[
        pltpu.VMEM((NCHUNK, CHUNK), jnp.int32),
        pltpu.VMEM((NIDXL // CHUNK, CHUNK), jnp.int32),
        pltpu.VMEM((CHUNK, LANES), jnp.float32),
        pltpu.VMEM((RPT, LANES), jnp.float32),
        pltpu.VMEM_SHARED((NPAD, LANES), jnp.float32),
        pltpu.VMEM_SHARED((NPAD, LANES), jnp.float32),
        pltpu.SemaphoreType.DMA,
    ],
  )


DEPTH = 3


def _sc_spmm_body(hws_hbm, src_hbm, dst_hbm, z_hbm, out_hbm,
                  src_v, dst_v, rows_v, acc_sp, gsem, ssem):
    c = lax.axis_index("c")
    s = lax.axis_index("s")
    wid = s * NC + c

    pltpu.sync_copy(z_hbm, acc_sp.at[pl.ds(s * RPT, RPT)])
    plsc.subcore_barrier()

    pltpu.sync_copy(src_hbm.at[wid], src_v)
    pltpu.sync_copy(dst_hbm.at[wid], dst_v)

    for p in range(DEPTH - 1):
        pltpu.async_copy(hws_hbm.at[src_v.at[p]], rows_v.at[p], gsem)

    def chunk(j, carry):
        @pl.when(j >= 1)
        def _():
            pltpu.make_async_copy(rows_v.at[(j - 1) % DEPTH],
                                  acc_sp.at[dst_v.at[j - 1]], ssem).wait()

        nxt = j + DEPTH - 1

        @pl.when(nxt < NCHUNK)
        def _():
            pltpu.async_copy(hws_hbm.at[src_v.at[nxt]],
                             rows_v.at[nxt % DEPTH], gsem)

        pltpu.make_async_copy(hws_hbm.at[src_v.at[j]], rows_v.at[j % DEPTH],
                              gsem).wait()
        pltpu.async_copy(rows_v.at[j % DEPTH], acc_sp.at[dst_v.at[j]], ssem,
                         add=True)
        return carry

    lax.fori_loop(0, NCHUNK, chunk, 0)
    pltpu.make_async_copy(rows_v.at[(NCHUNK - 1) % DEPTH],
                          acc_sp.at[dst_v.at[NCHUNK - 1]], ssem).wait()

    plsc.subcore_barrier()
    pltpu.sync_copy(acc_sp.at[pl.ds(s * RPT, RPT)],
                    out_hbm.at[pl.ds((c * NS + s) * RPT, RPT)])


@functools.cache
def _get_sc_spmm():
  return pl.kernel(
    _sc_spmm_body,
    out_type=jax.ShapeDtypeStruct((NC * NPAD, NHID), jnp.float32),
    mesh=plsc.VectorSubcoreMesh(core_axis_name="c", subcore_axis_name="s",
                                num_cores=NC, num_subcores=NS),
    compiler_params=pltpu.CompilerParams(use_tc_tiling_on_sc=False),
    scratch_types=[
        pltpu.VMEM((NCHUNK, CHUNK), jnp.int32),
        pltpu.VMEM((NCHUNK, CHUNK), jnp.int32),
        pltpu.VMEM((DEPTH, CHUNK, NHID), jnp.float32),
        pltpu.VMEM_SHARED((NPAD, NHID), jnp.float32),
        pltpu.SemaphoreType.DMA,
        pltpu.SemaphoreType.DMA,
    ],
  )


BM = 512
GRID = NPAD // BM


def _tc1_body(x_ref, y_ref, mark_ref, dega_ref, degb_ref, w1a_ref, w1b_ref,
              hws_ref, sl_ref, dinv_ref, dself_ref):
    deg = dega_ref[...] + degb_ref[...] + 2.0
    dinv = lax.rsqrt(deg)
    dself = 2.0 / deg
    oh = (y_ref[...] == lax.broadcasted_iota(
        jnp.int32, (BM, NLABEL), 1)).astype(jnp.float32)
    oh = oh * jnp.minimum(mark_ref[...], 1.0)
    hw = (jnp.dot(x_ref[...], w1a_ref[...],
                  preferred_element_type=jnp.float32)
          + jnp.dot(oh, w1b_ref[...], preferred_element_type=jnp.float32))
    hws_ref[...] = dinv * hw
    sl_ref[...] = dself * hw
    dinv_ref[...] = dinv
    dself_ref[...] = dself


_tc1 = pl.pallas_call(
    _tc1_body,
    grid=(GRID,),
    in_specs=[
        pl.BlockSpec((BM, NFEAT), lambda i: (i, 0)),
        pl.BlockSpec((BM, 1), lambda i: (i, 0)),
        pl.BlockSpec((BM, 1), lambda i: (i, 0)),
        pl.BlockSpec((BM, 1), lambda i: (i, 0)),
        pl.BlockSpec((BM, 1), lambda i: (i, 0)),
        pl.BlockSpec((NFEAT, NHID), lambda i: (0, 0)),
        pl.BlockSpec((NLABEL, NHID), lambda i: (0, 0)),
    ],
    out_specs=[
        pl.BlockSpec((BM, NHID), lambda i: (i, 0)),
        pl.BlockSpec((BM, NHID), lambda i: (i, 0)),
        pl.BlockSpec((BM, 1), lambda i: (i, 0)),
        pl.BlockSpec((BM, 1), lambda i: (i, 0)),
    ],
    out_shape=[
        jax.ShapeDtypeStruct((NPAD, NHID), jnp.float32),
        jax.ShapeDtypeStruct((NPAD, NHID), jnp.float32),
        jax.ShapeDtypeStruct((NPAD, 1), jnp.float32),
        jax.ShapeDtypeStruct((NPAD, 1), jnp.float32),
    ],
)


def _tc2_body(acca_ref, accb_ref, sl_ref, dinv_ref, dself_ref, b_ref, w_ref,
              hws_ref, slo_ref):
    dinv = dinv_ref[...]
    h = dinv * (acca_ref[...] + accb_ref[...]) + sl_ref[...] + b_ref[...]
    h = jnp.maximum(h, 0.0)
    hw = jnp.dot(h, w_ref[...], preferred_element_type=jnp.float32)
    hws_ref[...] = dinv * hw
    slo_ref[...] = dself_ref[...] * hw


_tc2 = pl.pallas_call(
    _tc2_body,
    grid=(GRID,),
    in_specs=[
        pl.BlockSpec((BM, NHID), lambda i: (i, 0)),
        pl.BlockSpec((BM, NHID), lambda i: (i, 0)),
        pl.BlockSpec((BM, NHID), lambda i: (i, 0)),
        pl.BlockSpec((BM, 1), lambda i: (i, 0)),
        pl.BlockSpec((BM, 1), lambda i: (i, 0)),
        pl.BlockSpec((1, NHID), lambda i: (0, 0)),
        pl.BlockSpec((NHID, NHID), lambda i: (0, 0)),
    ],
    out_specs=[
        pl.BlockSpec((BM, NHID), lambda i: (i, 0)),
        pl.BlockSpec((BM, NHID), lambda i: (i, 0)),
    ],
    out_shape=[
        jax.ShapeDtypeStruct((NPAD, NHID), jnp.float32),
        jax.ShapeDtypeStruct((NPAD, NHID), jnp.float32),
    ],
)


def _tc3_body(acca_ref, accb_ref, sl_ref, dinv_ref, b2_ref,
              wf1_ref, bf1_ref, wf2_ref, bf2_ref, out_ref):
    h = dinv_ref[...] * (acca_ref[...] + accb_ref[...]) \
        + sl_ref[...] + b2_ref[...]
    h = jnp.maximum(h, 0.0)
    m = jnp.dot(h, wf1_ref[...], preferred_element_type=jnp.float32) \
        + bf1_ref[...]
    m = jnp.where(m > 0.0, m, jnp.exp(jnp.minimum(m, 0.0)) - 1.0)
    logits = jnp.dot(m, wf2_ref[...], preferred_element_type=jnp.float32) \
        + bf2_ref[...]
    mx = jnp.max(logits, axis=1, keepdims=True)
    sh = logits - mx
    lse = jnp.log(jnp.sum(jnp.exp(sh), axis=1, keepdims=True))
    out_ref[...] = sh - lse


_tc3 = pl.pallas_call(
    _tc3_body,
    grid=(GRID,),
    in_specs=[
        pl.BlockSpec((BM, NHID), lambda i: (i, 0)),
        pl.BlockSpec((BM, NHID), lambda i: (i, 0)),
        pl.BlockSpec((BM, NHID), lambda i: (i, 0)),
        pl.BlockSpec((BM, 1), lambda i: (i, 0)),
        pl.BlockSpec((1, NHID), lambda i: (0, 0)),
        pl.BlockSpec((NHID, 2 * NHID), lambda i: (0, 0)),
        pl.BlockSpec((1, 2 * NHID), lambda i: (0, 0)),
        pl.BlockSpec((2 * NHID, NLABEL), lambda i: (0, 0)),
        pl.BlockSpec((1, NLABEL), lambda i: (0, 0)),
    ],
    out_specs=pl.BlockSpec((BM, NLABEL), lambda i: (i, 0)),
    out_shape=jax.ShapeDtypeStruct((NPAD, NLABEL), jnp.float32),
)


def kernel(x, y, predictions, adj, idx_labeled, n_sample,
           W1, b1, W2, b2, Wf1, bf1, Wf2, bf2):
    src = adj[0]
    dst = adj[1]
    pad_i = jnp.arange(EPAD - E, dtype=jnp.int32)
    src_p = jnp.concatenate(
        [src, pad_i % N]).reshape(NW, NCHUNK, CHUNK)
    dst_p = jnp.concatenate(
        [dst, N + pad_i % (NPAD - N)]).reshape(NW, NCHUNK, CHUNK)
    idxl_p = jnp.concatenate(
        [idx_labeled, jnp.full((NIDXL - idx_labeled.shape[0],), NPAD - 1,
                               jnp.int32)]).reshape(NIDXL // CHUNK, CHUNK)
    y_p = jnp.concatenate([y, jnp.zeros((NPAD - N,), jnp.int32)])
    x_p = jnp.pad(x, ((0, NPAD - N), (0, 0)))
    ones_le = jnp.ones((CHUNK, LANES), jnp.float32)
    z_le = jnp.zeros((RPT, LANES), jnp.float32)
    z_row = jnp.zeros((RPT, NHID), jnp.float32)

    deg_part, mark = _get_sc_deg_feats()(dst_p, idxl_p, ones_le, z_le)
    dega = deg_part[:NPAD, 0:1]
    degb = deg_part[NPAD:, 0:1]
    mark_col = mark[:NPAD, 0:1] + mark[NPAD:, 0:1]

    hws1, sl1, dinv, dself = _tc1(
        x_p, y_p.reshape(NPAD, 1), mark_col, dega, degb,
        W1[:NFEAT], W1[NFEAT:])

    acc1 = _get_sc_spmm()(hws1, src_p, dst_p, z_row)

    hws2, sl2 = _tc2(acc1[:NPAD], acc1[NPAD:], sl1, dinv, dself,
                     b1.reshape(1, NHID), W2)

    acc2 = _get_sc_spmm()(hws2, src_p, dst_p, z_row)

    out = _tc3(acc2[:NPAD], acc2[NPAD:], sl2, dinv, b2.reshape(1, NHID),
               Wf1, bf1.reshape(1, 2 * NHID), Wf2, bf2.reshape(1, NLABEL))
    return out[:N]


# speedup vs baseline: 1.0247x; 1.0207x over previous
import functools

import jax
import jax.numpy as jnp
from jax import lax
from jax.experimental import pallas as pl
from jax.experimental.pallas import tpu as pltpu
from jax.experimental.pallas import tpu_sc as plsc

N = 10000
E = 320000
NFEAT = 128
NLABEL = 16
NHID = 128

NC, NS, LANES = 2, 16, 16
NW = NC * NS
NPAD = 10240
CHUNK = 56
NCHUNK = (E + NW * CHUNK - 1) // (NW * CHUNK)
EPAD = NW * NCHUNK * CHUNK
RPT = NPAD // NS
NIDXL = 1008


def _sc_deg_feats_body(dst_hbm, idxl_hbm, ones_hbm, z_hbm,
                       deg_out, mark_out,
                       dst_v, idxl_v, ones_v, z_v, deg_sp, mark_sp, sem):
    c = lax.axis_index("c")
    s = lax.axis_index("s")
    wid = s * NC + c

    pltpu.sync_copy(z_hbm, z_v)
    pltpu.sync_copy(z_v, deg_sp.at[pl.ds(s * RPT, RPT)])
    pltpu.sync_copy(z_v, mark_sp.at[pl.ds(s * RPT, RPT)])
    plsc.subcore_barrier()

    pltpu.sync_copy(ones_hbm, ones_v)
    pltpu.sync_copy(dst_hbm.at[wid], dst_v)
    pltpu.sync_copy(idxl_hbm, idxl_v)

    def chunk(j, carry):
        pltpu.async_copy(ones_v, deg_sp.at[dst_v.at[j]], sem, add=True)
        return carry

    lax.fori_loop(0, NCHUNK, chunk, 0)

    for j in range(NIDXL // CHUNK):
        pltpu.async_copy(ones_v, mark_sp.at[idxl_v.at[j]], sem, add=True)

    def drain(j, carry):
        pltpu.make_async_copy(ones_v, deg_sp.at[dst_v.at[j]], sem).wait()
        return carry

    lax.fori_loop(0, NCHUNK, drain, 0)
    for j in range(NIDXL // CHUNK):
        pltpu.make_async_copy(ones_v, mark_sp.at[idxl_v.at[j]], sem).wait()

    plsc.subcore_barrier()
    pltpu.sync_copy(deg_sp.at[pl.ds(s * RPT, RPT)],
                    deg_out.at[pl.ds((c * NS + s) * RPT, RPT)])
    pltpu.sync_copy(mark_sp.at[pl.ds(s * RPT, RPT)],
                    mark_out.at[pl.ds((c * NS + s) * RPT, RPT)])


@functools.cache
def _get_sc_deg_feats():
  return pl.kernel(
    _sc_deg_feats_body,
    out_type=(jax.ShapeDtypeStruct((NC * NPAD, LANES), jnp.float32),
              jax.ShapeDtypeStruct((NC * NPAD, LANES), jnp.float32)),
    mesh=plsc.VectorSubcoreMesh(core_axis_name="c", subcore_axis_name="s",
                                num_cores=NC, num_subcores=NS),
    compiler_params=pltpu.CompilerParams(use_tc_tiling_on_sc=False),
    scratch_types=[
        pltpu.VMEM((NCHUNK, CHUNK), jnp.int32),
        pltpu.VMEM((NIDXL // CHUNK, CHUNK), jnp.int32),
        pltpu.VMEM((CHUNK, LANES), jnp.float32),
        pltpu.VMEM((RPT, LANES), jnp.float32),
        pltpu.VMEM_SHARED((NPAD, LANES), jnp.float32),
        pltpu.VMEM_SHARED((NPAD, LANES), jnp.float32),
        pltpu.SemaphoreType.DMA,
    ],
  )


DEPTH = 4


def _sc_spmm_body(hws_hbm, src_hbm, dst_hbm, z_hbm, out_hbm,
                  src_v, dst_v, rows_v, acc_sp, gsem, ssem):
    c = lax.axis_index("c")
    s = lax.axis_index("s")
    wid = s * NC + c

    pltpu.sync_copy(z_hbm, acc_sp.at[pl.ds(s * RPT, RPT)])
    plsc.subcore_barrier()

    pltpu.sync_copy(src_hbm.at[wid], src_v)
    pltpu.sync_copy(dst_hbm.at[wid], dst_v)

    for p in range(DEPTH - 1):
        pltpu.async_copy(hws_hbm.at[src_v.at[p]], rows_v.at[p], gsem)

    def chunk(j, carry):
        @pl.when(j >= 1)
        def _():
            pltpu.make_async_copy(rows_v.at[(j - 1) % DEPTH],
                                  acc_sp.at[dst_v.at[j - 1]], ssem).wait()

        nxt = j + DEPTH - 1

        @pl.when(nxt < NCHUNK)
        def _():
            pltpu.async_copy(hws_hbm.at[src_v.at[nxt]],
                             rows_v.at[nxt % DEPTH], gsem)

        pltpu.make_async_copy(hws_hbm.at[src_v.at[j]], rows_v.at[j % DEPTH],
                              gsem).wait()
        pltpu.async_copy(rows_v.at[j % DEPTH], acc_sp.at[dst_v.at[j]], ssem,
                         add=True)
        return carry

    lax.fori_loop(0, NCHUNK, chunk, 0)
    pltpu.make_async_copy(rows_v.at[(NCHUNK - 1) % DEPTH],
                          acc_sp.at[dst_v.at[NCHUNK - 1]], ssem).wait()

    plsc.subcore_barrier()
    pltpu.sync_copy(acc_sp.at[pl.ds(s * RPT, RPT)],
                    out_hbm.at[pl.ds((c * NS + s) * RPT, RPT)])


@functools.cache
def _get_sc_spmm():
  return pl.kernel(
    _sc_spmm_body,
    out_type=jax.ShapeDtypeStruct((NC * NPAD, NHID), jnp.float32),
    mesh=plsc.VectorSubcoreMesh(core_axis_name="c", subcore_axis_name="s",
                                num_cores=NC, num_subcores=NS),
    compiler_params=pltpu.CompilerParams(use_tc_tiling_on_sc=False),
    scratch_types=[
        pltpu.VMEM((NCHUNK, CHUNK), jnp.int32),
        pltpu.VMEM((NCHUNK, CHUNK), jnp.int32),
        pltpu.VMEM((DEPTH, CHUNK, NHID), jnp.float32),
        pltpu.VMEM_SHARED((NPAD, NHID), jnp.float32),
        pltpu.SemaphoreType.DMA,
        pltpu.SemaphoreType.DMA,
    ],
  )


BM = 512
GRID = NPAD // BM


def _tc1_body(x_ref, y_ref, mark_ref, dega_ref, degb_ref, w1a_ref, w1b_ref,
              hws_ref, sl_ref, dinv_ref, dself_ref):
    deg = dega_ref[...] + degb_ref[...] + 2.0
    dinv = lax.rsqrt(deg)
    dself = 2.0 / deg
    oh = (y_ref[...] == lax.broadcasted_iota(
        jnp.int32, (BM, NLABEL), 1)).astype(jnp.float32)
    oh = oh * jnp.minimum(mark_ref[...], 1.0)
    hw = (jnp.dot(x_ref[...], w1a_ref[...],
                  preferred_element_type=jnp.float32)
          + jnp.dot(oh, w1b_ref[...], preferred_element_type=jnp.float32))
    hws_ref[...] = dinv * hw
    sl_ref[...] = dself * hw
    dinv_ref[...] = dinv
    dself_ref[...] = dself


_tc1 = pl.pallas_call(
    _tc1_body,
    grid=(GRID,),
    in_specs=[
        pl.BlockSpec((BM, NFEAT), lambda i: (i, 0)),
        pl.BlockSpec((BM, 1), lambda i: (i, 0)),
        pl.BlockSpec((BM, 1), lambda i: (i, 0)),
        pl.BlockSpec((BM, 1), lambda i: (i, 0)),
        pl.BlockSpec((BM, 1), lambda i: (i, 0)),
        pl.BlockSpec((NFEAT, NHID), lambda i: (0, 0)),
        pl.BlockSpec((NLABEL, NHID), lambda i: (0, 0)),
    ],
    out_specs=[
        pl.BlockSpec((BM, NHID), lambda i: (i, 0)),
        pl.BlockSpec((BM, NHID), lambda i: (i, 0)),
        pl.BlockSpec((BM, 1), lambda i: (i, 0)),
        pl.BlockSpec((BM, 1), lambda i: (i, 0)),
    ],
    out_shape=[
        jax.ShapeDtypeStruct((NPAD, NHID), jnp.float32),
        jax.ShapeDtypeStruct((NPAD, NHID), jnp.float32),
        jax.ShapeDtypeStruct((NPAD, 1), jnp.float32),
        jax.ShapeDtypeStruct((NPAD, 1), jnp.float32),
    ],
)


def _tc2_body(acca_ref, accb_ref, sl_ref, dinv_ref, dself_ref, b_ref, w_ref,
              hws_ref, slo_ref):
    dinv = dinv_ref[...]
    h = dinv * (acca_ref[...] + accb_ref[...]) + sl_ref[...] + b_ref[...]
    h = jnp.maximum(h, 0.0)
    hw = jnp.dot(h, w_ref[...], preferred_element_type=jnp.float32)
    hws_ref[...] = dinv * hw
    slo_ref[...] = dself_ref[...] * hw


_tc2 = pl.pallas_call(
    _tc2_body,
    grid=(GRID,),
    in_specs=[
        pl.BlockSpec((BM, NHID), lambda i: (i, 0)),
        pl.BlockSpec((BM, NHID), lambda i: (i, 0)),
        pl.BlockSpec((BM, NHID), lambda i: (i, 0)),
        pl.BlockSpec((BM, 1), lambda i: (i, 0)),
        pl.BlockSpec((BM, 1), lambda i: (i, 0)),
        pl.BlockSpec((1, NHID), lambda i: (0, 0)),
        pl.BlockSpec((NHID, NHID), lambda i: (0, 0)),
    ],
    out_specs=[
        pl.BlockSpec((BM, NHID), lambda i: (i, 0)),
        pl.BlockSpec((BM, NHID), lambda i: (i, 0)),
    ],
    out_shape=[
        jax.ShapeDtypeStruct((NPAD, NHID), jnp.float32),
        jax.ShapeDtypeStruct((NPAD, NHID), jnp.float32),
    ],
)


def _tc3_body(acca_ref, accb_ref, sl_ref, dinv_ref, b2_ref,
              wf1_ref, bf1_ref, wf2_ref, bf2_ref, out_ref):
    h = dinv_ref[...] * (acca_ref[...] + accb_ref[...]) \
        + sl_ref[...] + b2_ref[...]
    h = jnp.maximum(h, 0.0)
    m = jnp.dot(h, wf1_ref[...], preferred_element_type=jnp.float32) \
        + bf1_ref[...]
    m = jnp.where(m > 0.0, m, jnp.exp(jnp.minimum(m, 0.0)) - 1.0)
    logits = jnp.dot(m, wf2_ref[...], preferred_element_type=jnp.float32) \
        + bf2_ref[...]
    mx = jnp.max(logits, axis=1, keepdims=True)
    sh = logits - mx
    lse = jnp.log(jnp.sum(jnp.exp(sh), axis=1, keepdims=True))
    out_ref[...] = sh - lse


_tc3 = pl.pallas_call(
    _tc3_body,
    grid=(GRID,),
    in_specs=[
        pl.BlockSpec((BM, NHID), lambda i: (i, 0)),
        pl.BlockSpec((BM, NHID), lambda i: (i, 0)),
        pl.BlockSpec((BM, NHID), lambda i: (i, 0)),
        pl.BlockSpec((BM, 1), lambda i: (i, 0)),
        pl.BlockSpec((1, NHID), lambda i: (0, 0)),
        pl.BlockSpec((NHID, 2 * NHID), lambda i: (0, 0)),
        pl.BlockSpec((1, 2 * NHID), lambda i: (0, 0)),
        pl.BlockSpec((2 * NHID, NLABEL), lambda i: (0, 0)),
        pl.BlockSpec((1, NLABEL), lambda i: (0, 0)),
    ],
    out_specs=pl.BlockSpec((BM, NLABEL), lambda i: (i, 0)),
    out_shape=jax.ShapeDtypeStruct((NPAD, NLABEL), jnp.float32),
)


def kernel(x, y, predictions, adj, idx_labeled, n_sample,
           W1, b1, W2, b2, Wf1, bf1, Wf2, bf2):
    src = adj[0]
    dst = adj[1]
    pad_i = jnp.arange(EPAD - E, dtype=jnp.int32)
    src_p = jnp.concatenate(
        [src, pad_i % N]).reshape(NW, NCHUNK, CHUNK)
    dst_p = jnp.concatenate(
        [dst, N + pad_i % (NPAD - N)]).reshape(NW, NCHUNK, CHUNK)
    idxl_p = jnp.concatenate(
        [idx_labeled, jnp.full((NIDXL - idx_labeled.shape[0],), NPAD - 1,
                               jnp.int32)]).reshape(NIDXL // CHUNK, CHUNK)
    y_p = jnp.concatenate([y, jnp.zeros((NPAD - N,), jnp.int32)])
    x_p = jnp.pad(x, ((0, NPAD - N), (0, 0)))
    ones_le = jnp.ones((CHUNK, LANES), jnp.float32)
    z_le = jnp.zeros((RPT, LANES), jnp.float32)
    z_row = jnp.zeros((RPT, NHID), jnp.float32)

    deg_part, mark = _get_sc_deg_feats()(dst_p, idxl_p, ones_le, z_le)
    dega = deg_part[:NPAD, 0:1]
    degb = deg_part[NPAD:, 0:1]
    mark_col = mark[:NPAD, 0:1] + mark[NPAD:, 0:1]

    hws1, sl1, dinv, dself = _tc1(
        x_p, y_p.reshape(NPAD, 1), mark_col, dega, degb,
        W1[:NFEAT], W1[NFEAT:])

    acc1 = _get_sc_spmm()(hws1, src_p, dst_p, z_row)

    hws2, sl2 = _tc2(acc1[:NPAD], acc1[NPAD:], sl1, dinv, dself,
                     b1.reshape(1, NHID), W2)

    acc2 = _get_sc_spmm()(hws2, src_p, dst_p, z_row)

    out = _tc3(acc2[:NPAD], acc2[NPAD:], sl2, dinv, b2.reshape(1, NHID),
               Wf1, bf1.reshape(1, 2 * NHID), Wf2, bf2.reshape(1, NLABEL))
    return out[:N]
